# Initial kernel scaffold; baseline (speedup 1.0000x reference)
import jax
import jax.numpy as jnp
from jax.experimental import pallas as pl


def kernel(vertices):
    raise NotImplementedError("write your pallas kernel here")


# speedup vs baseline: 21.7369x; 21.7369x over previous
import functools

import jax
import jax.numpy as jnp
from jax.experimental import pallas as pl

K = 16
AZB = 8
ELB = 2
RADB = 2
HB = 11
TOTAL = AZB * ELB * RADB * HB
N = 2048
R = 256
BIG = 3.4e38

_INTERPRET = False


def _rot(a, v, p, q):
    app, aqq, apq = a[p][p], a[q][q], a[p][q]
    tau = (app - aqq) / (2.0 * apq)
    t = jnp.sign(tau) / (jnp.abs(tau) + jnp.sqrt(1.0 + tau * tau))
    t = jnp.where(tau == 0.0, 1.0, t)
    t = jnp.where(apq == 0.0, 0.0, t)
    c = jax.lax.rsqrt(1.0 + t * t)
    s = t * c
    rowp = [a[p][j] for j in range(3)]
    rowq = [a[q][j] for j in range(3)]
    for j in range(3):
        a[p][j] = c * rowp[j] + s * rowq[j]
        a[q][j] = -s * rowp[j] + c * rowq[j]
    colp = [a[i][p] for i in range(3)]
    colq = [a[i][q] for i in range(3)]
    for i in range(3):
        a[i][p] = c * colp[i] + s * colq[i]
        a[i][q] = -s * colp[i] + c * colq[i]
    for i in range(3):
        vip, viq = v[i][p], v[i][q]
        v[i][p] = c * vip + s * viq
        v[i][q] = -s * vip + c * viq


def _stage1_kernel(v_ref, vt_ref, normt_ref, localt_ref, idxt_ref, maxn_ref):
    t = pl.program_id(1)
    V = v_ref[0]
    VT = vt_ref[0]
    VrT = vt_ref[0, :, pl.ds(t * R, R)]
    sqa = jnp.sum(V * V, axis=1, keepdims=True)
    sqr = jnp.sum(VrT * VrT, axis=0, keepdims=True)
    cross = jnp.dot(V, VrT, preferred_element_type=jnp.float32)
    d2 = sqa + sqr - 2.0 * cross

    iota_s = jax.lax.broadcasted_iota(jnp.int32, (N, R), 0)
    vals = d2
    idx_rows = []
    nb_rows = []
    for _ in range(K):
        m = jnp.min(vals, axis=0, keepdims=True)
        eq = vals == m
        a = jnp.min(jnp.where(eq, iota_s, N), axis=0, keepdims=True)
        oh = iota_s == a
        idx_rows.append(a)
        ohf = oh.astype(jnp.float32)
        nb = jnp.dot(VT, ohf, preferred_element_type=jnp.float32,
                     precision=jax.lax.Precision.HIGHEST)
        nb_rows.append(nb)
        vals = jnp.where(oh, BIG, vals)

    idxT = jnp.concatenate(idx_rows, axis=0)
    nx = jnp.concatenate([nb[0:1, :] - VrT[0:1, :] for nb in nb_rows], axis=0)
    ny = jnp.concatenate([nb[1:2, :] - VrT[1:2, :] for nb in nb_rows], axis=0)
    nz = jnp.concatenate([nb[2:3, :] - VrT[2:3, :] for nb in nb_rows], axis=0)

    dist = jnp.sqrt(jnp.maximum(nx * nx + ny * ny + nz * nz, 1e-12))
    rmax = jnp.max(dist, axis=0, keepdims=True)
    w = jnp.maximum(rmax - dist, 0.0)
    sw = jnp.maximum(jnp.sum(w, axis=0, keepdims=True), 1e-12)

    def _bf(u):
        return u.astype(jnp.bfloat16).astype(jnp.float32)

    bwx, bwy, bwz = _bf(w * nx), _bf(w * ny), _bf(w * nz)
    bx, by, bz = _bf(nx), _bf(ny), _bf(nz)

    def _e(u, vv):
        return jnp.sum(u * vv, axis=0, keepdims=True) / sw

    cxx = _e(bwx, bx)
    cyy = _e(bwy, by)
    czz = _e(bwz, bz)
    cxy = (_e(bwx, by) + _e(bwy, bx)) * 0.5
    cxz = (_e(bwx, bz) + _e(bwz, bx)) * 0.5
    cyz = (_e(bwy, bz) + _e(bwz, by)) * 0.5

    a = [[cxx, cxy, cxz], [cxy, cyy, cyz], [cxz, cyz, czz]]
    one = jnp.ones_like(cxx)
    zero = jnp.zeros_like(cxx)
    v = [[one, zero, zero], [zero, one, zero], [zero, zero, one]]
    for _ in range(6):
        _rot(a, v, 0, 2)
        _rot(a, v, 1, 2)
        _rot(a, v, 0, 1)

    l0, l1, l2 = a[0][0], a[1][1], a[2][2]

    c01 = l0 <= l1
    vmin = [jnp.where(c01, v[i][0], v[i][1]) for i in range(3)]
    lmin = jnp.where(c01, l0, l1)
    cmin = lmin <= l2
    zv = [jnp.where(cmin, vmin[i], v[i][2]) for i in range(3)]

    d01 = l0 > l1
    vmax = [jnp.where(d01, v[i][0], v[i][1]) for i in range(3)]
    lmax = jnp.where(d01, l0, l1)
    cmax = lmax > l2
    xv = [jnp.where(cmax, vmax[i], v[i][2]) for i in range(3)]

    dotx = bx * _bf(xv[0]) + by * _bf(xv[1]) + bz * _bf(xv[2])
    sx = jnp.sum((dotx >= 0.0).astype(jnp.float32), axis=0, keepdims=True)
    flipx = 2.0 * sx < K
    xs = [jnp.where(flipx, -xv[i], xv[i]) for i in range(3)]

    dotz = bx * _bf(zv[0]) + by * _bf(zv[1]) + bz * _bf(zv[2])
    sz = jnp.sum((dotz >= 0.0).astype(jnp.float32), axis=0, keepdims=True)
    flipz = 2.0 * sz < K
    zs = [jnp.where(flipz, -zv[i], zv[i]) for i in range(3)]

    y0 = zs[1] * xs[2] - zs[2] * xs[1]
    y1 = zs[2] * xs[0] - zs[0] * xs[2]
    y2 = zs[0] * xs[1] - zs[1] * xs[0]
    lx = jnp.where(flipx, -dotx, dotx)
    ly = nx * y0 + ny * y1 + nz * y2
    lz = jnp.where(flipz, -dotz, dotz)
    zero_self = (nx == 0.0) & (ny == 0.0) & (nz == 0.0)
    lx = jnp.where(zero_self, 0.0, lx)
    ly = jnp.where(zero_self, 0.0, ly)
    lz = jnp.where(zero_self, 0.0, lz)

    normt_ref[0] = jnp.concatenate([zs[0], zs[1], zs[2]], axis=0)
    localt_ref[0, 0] = lx
    localt_ref[0, 1] = ly
    localt_ref[0, 2] = lz
    idxt_ref[0] = idxT
    nrm = jnp.sqrt(lx * lx + ly * ly + lz * lz)
    maxn_ref[0, 0] = jnp.broadcast_to(jnp.max(nrm), (1, 128))


def _stage2_kernel(normt_ref, localt_ref, idxt_ref, rad_ref, desct_ref):
    t = pl.program_id(1)
    NT = normt_ref[0]
    lx = localt_ref[0, 0]
    ly = localt_ref[0, 1]
    lz = localt_ref[0, 2]
    idxT = idxt_ref[0]
    rad = rad_ref[0, 0, 0]
    centerT = normt_ref[0, :, pl.ds(t * R, R)]

    def _bf(u):
        return u.astype(jnp.bfloat16).astype(jnp.float32)

    bc = _bf(centerT)
    iota_s = jax.lax.broadcasted_iota(jnp.int32, (N, R), 0)
    cos_rows = []
    for kk in range(K):
        oh = (iota_s == idxT[kk:kk + 1, :]).astype(jnp.float32)
        nn = jnp.dot(NT, oh, preferred_element_type=jnp.float32,
                     precision=jax.lax.Precision.HIGHEST)
        bn = _bf(nn)
        cos_rows.append((bn[0:1, :] * bc[0:1, :] + bn[1:2, :] * bc[1:2, :])
                        + bn[2:3, :] * bc[2:3, :])
    cosang = jnp.clip(jnp.concatenate(cos_rows, axis=0), -1.0, 1.0)

    hist_bin = jnp.clip(jnp.floor((cosang + 1.0) * 0.5 * HB).astype(jnp.int32),
                        0, HB - 1)
    dist = jnp.sqrt(jnp.maximum(lx * lx + ly * ly + lz * lz, 1e-12))
    az = jnp.arctan2(ly, lx)
    az_bin = jnp.clip(
        jnp.floor((az + jnp.pi) / (2.0 * jnp.pi) * AZB).astype(jnp.int32),
        0, AZB - 1)
    cos_el = jnp.clip(lz / dist, -1.0, 1.0)
    el_bin = jnp.clip(jnp.floor((cos_el + 1.0) * 0.5 * ELB).astype(jnp.int32),
                      0, ELB - 1)
    rad_bin = jnp.clip(
        jnp.floor(dist / jnp.maximum(rad, 1e-12) * RADB).astype(jnp.int32),
        0, RADB - 1)
    flat = ((az_bin * ELB + el_bin) * RADB + rad_bin) * HB + hist_bin

    iota_b = jax.lax.broadcasted_iota(jnp.int32, (TOTAL, R), 0)
    acc = jnp.zeros((TOTAL, R), jnp.float32)
    for kk in range(K):
        acc = acc + (iota_b == flat[kk:kk + 1, :]).astype(jnp.float32)
    norm = jnp.sqrt(jnp.sum(acc * acc, axis=0, keepdims=True))
    desct_ref[0] = acc / jnp.maximum(norm, 1e-12)


@jax.jit
def kernel(vertices):
    B = vertices.shape[0]
    nt = N // R
    vT = jnp.swapaxes(vertices, 1, 2)

    normT, localT, idxT, maxn = pl.pallas_call(
        _stage1_kernel,
        grid=(B, nt),
        in_specs=[
            pl.BlockSpec((1, N, 3), lambda b, t: (b, 0, 0)),
            pl.BlockSpec((1, 3, N), lambda b, t: (b, 0, 0)),
        ],
        out_specs=[
            pl.BlockSpec((1, 3, R), lambda b, t: (b, 0, t)),
            pl.BlockSpec((1, 3, K, R), lambda b, t: (b, 0, 0, t)),
            pl.BlockSpec((1, K, R), lambda b, t: (b, 0, t)),
            pl.BlockSpec((1, 1, 1, 128), lambda b, t: (b, t, 0, 0)),
        ],
        out_shape=[
            jax.ShapeDtypeStruct((B, 3, N), jnp.float32),
            jax.ShapeDtypeStruct((B, 3, K, N), jnp.float32),
            jax.ShapeDtypeStruct((B, K, N), jnp.int32),
            jax.ShapeDtypeStruct((B, nt, 1, 128), jnp.float32),
        ],
        interpret=_INTERPRET,
    )(vertices, vT)

    radius = jnp.max(maxn, axis=(1, 2, 3))
    rad_in = jnp.broadcast_to(radius[:, None, None], (B, 1, 128))

    descT = pl.pallas_call(
        _stage2_kernel,
        grid=(B, nt),
        in_specs=[
            pl.BlockSpec((1, 3, N), lambda b, t: (b, 0, 0)),
            pl.BlockSpec((1, 3, K, R), lambda b, t: (b, 0, 0, t)),
            pl.BlockSpec((1, K, R), lambda b, t: (b, 0, t)),
            pl.BlockSpec((1, 1, 128), lambda b, t: (b, 0, 0)),
        ],
        out_specs=pl.BlockSpec((1, TOTAL, R), lambda b, t: (b, 0, t)),
        out_shape=jax.ShapeDtypeStruct((B, TOTAL, N), jnp.float32),
        interpret=_INTERPRET,
    )(normT, localT, idxT, rad_in)

    return jnp.swapaxes(descT, 1, 2)
